# baseline (speedup 1.0000x reference)
import functools

import jax
import jax.numpy as jnp
from jax import lax
from jax.experimental import pallas as pl
from jax.experimental.pallas import tpu as pltpu
from jax.experimental.pallas import tpu_sc as plsc

_NUM_CORES = 2
_NUM_SUBCORES = 16
_NUM_WORKERS = _NUM_CORES * _NUM_SUBCORES


@functools.partial(jax.jit, static_argnums=(2,))
def _sc_gather(idx_ts, weight, T):
    S = idx_ts.shape[1]
    D = weight.shape[1]
    SB = S // 128
    sb_per_w = SB // _NUM_WORKERS
    n_units = T * sb_per_w
    mesh = plsc.VectorSubcoreMesh(core_axis_name="c", subcore_axis_name="s")

    @functools.partial(
        pl.kernel,
        mesh=mesh,
        out_type=jax.ShapeDtypeStruct((T, D // 8, SB, 8, 128), jnp.float32),
        scratch_types=[
            pltpu.VMEM((T, sb_per_w * 128), jnp.int32),
            pltpu.VMEM((128, D), jnp.float32),
            pltpu.VMEM((128, D), jnp.float32),
            pltpu.VMEM((D // 8, 8, 128), jnp.float32),
            pltpu.VMEM((D // 8, 8, 128), jnp.float32),
            pltpu.SemaphoreType.DMA,
            pltpu.SemaphoreType.DMA,
            pltpu.SemaphoreType.DMA,
            pltpu.SemaphoreType.DMA,
        ],
        compiler_params=pltpu.CompilerParams(
            use_tc_tiling_on_sc=False, needs_layout_passes=False),
    )
    def k(idx_hbm, table_hbm, out_hbm, idx_v, rows0, rows1, trv0, trv1,
          g0, g1, w0, w1):
        wid = lax.axis_index("s") * _NUM_CORES + lax.axis_index("c")
        rows = (rows0, rows1)
        trv = (trv0, trv1)
        gsem = (g0, g1)
        wsem = (w0, w1)

        pltpu.sync_copy(idx_hbm.at[:, pl.ds(wid * sb_per_w * 128,
                                            sb_per_w * 128)], idx_v)

        lane = lax.iota(jnp.int32, 16)
        slvecs = [lane + 16 * b for b in range(8)]

        def unit_tj(u):
            t = u // sb_per_w
            j = u % sb_per_w
            return t, j

        def start_gather(u, p):
            t, j = unit_tj(u)
            pltpu.async_copy(
                table_hbm.at[idx_v.at[t, pl.ds(j * 128, 128)]],
                rows[p], gsem[p])

        def wait_gather(u, p):
            t, j = unit_tj(u)
            pltpu.make_async_copy(
                table_hbm.at[idx_v.at[t, pl.ds(j * 128, 128)]],
                rows[p], gsem[p]).wait()

        def out_slice(u):
            t, j = unit_tj(u)
            sb = wid * sb_per_w + j
            return out_hbm.at[t, :, sb]

        def start_write(u, p):
            pltpu.async_copy(trv[p], out_slice(u), wsem[p])

        def wait_write(u, p):
            pltpu.make_async_copy(trv[p], out_slice(u), wsem[p]).wait()

        def transpose(p):
            def cb_body(cb, carry):
                for cl in range(8):
                    c = cb * 8 + cl
                    cvec = lane * 0 + c
                    for b in range(8):
                        vals = plsc.load_gather(rows[p], [slvecs[b], cvec])
                        trv[p][cb, cl, pl.ds(b * 16, 16)] = vals
                return carry
            lax.fori_loop(0, D // 8, cb_body, 0)

        start_gather(0, 0)
        start_gather(1, 1)

        def body(i, carry):
            for p in range(2):
                u = 2 * i + p
                wait_gather(u, p)

                @pl.when(u >= 2)
                def _():
                    wait_write(u - 2, p)

                transpose(p)

                @pl.when(u + 2 < n_units)
                def _():
                    start_gather(u + 2, p)

                start_write(u, p)
            return carry

        lax.fori_loop(0, n_units // 2, body, 0)
        wait_write(n_units - 2, 0)
        wait_write(n_units - 1, 1)

    return k(idx_ts, weight)


def kernel(token_ids, weight):
    S, T = token_ids.shape
    D = weight.shape[1]
    idx_ts = token_ids.T.astype(jnp.int32)
    a = _sc_gather(idx_ts, weight, T)
    return a.transpose(2, 4, 0, 1, 3).reshape(S, T, D)


# speedup vs baseline: 1.1214x; 1.1214x over previous
import functools

import jax
import jax.numpy as jnp
from jax import lax
from jax.experimental import pallas as pl
from jax.experimental.pallas import tpu as pltpu
from jax.experimental.pallas import tpu_sc as plsc

_NUM_CORES = 2
_NUM_SUBCORES = 16
_NUM_WORKERS = _NUM_CORES * _NUM_SUBCORES


@functools.partial(jax.jit, static_argnums=(2,))
def _sc_gather(idx_ts, weight, T):
    S = idx_ts.shape[1]
    D = weight.shape[1]
    SB = S // 128
    JW = SB // _NUM_WORKERS
    R = JW * 128
    mesh = plsc.VectorSubcoreMesh(core_axis_name="c", subcore_axis_name="s")

    @functools.partial(
        pl.kernel,
        mesh=mesh,
        out_type=jax.ShapeDtypeStruct((T, D // 8, SB, 8, 128), jnp.float32),
        scratch_types=[
            pltpu.VMEM((R,), jnp.int32),
            pltpu.VMEM((R,), jnp.int32),
            pltpu.VMEM((R, D), jnp.float32),
            pltpu.VMEM((R, D), jnp.float32),
            pltpu.VMEM((D // 8, JW, 8, 128), jnp.float32),
            pltpu.SemaphoreType.DMA,
            pltpu.SemaphoreType.DMA,
            pltpu.SemaphoreType.DMA,
        ],
        compiler_params=pltpu.CompilerParams(
            use_tc_tiling_on_sc=False, needs_layout_passes=False),
    )
    def k(idx_hbm, table_hbm, out_hbm, ib0, ib1, rows0, rows1, trv,
          g0, g1, ws):
        wid = lax.axis_index("s") * _NUM_CORES + lax.axis_index("c")
        sb0 = wid * JW
        ibuf = (ib0, ib1)
        rows = (rows0, rows1)
        gsem = (g0, g1)

        lane = lax.iota(jnp.int32, 16)
        slvecs = [lane + 16 * b for b in range(8)]

        def load_idx(t, p):
            pltpu.sync_copy(idx_hbm.at[t, pl.ds(sb0 * 128, R)], ibuf[p])

        def start_gather(p):
            pltpu.async_copy(table_hbm.at[ibuf[p]], rows[p], gsem[p])

        def wait_gather(p):
            pltpu.make_async_copy(table_hbm.at[ibuf[p]], rows[p],
                                  gsem[p]).wait()

        def start_write(t):
            for cb in range(D // 8):
                pltpu.async_copy(trv.at[cb],
                                 out_hbm.at[t, cb, pl.ds(sb0, JW)], ws)

        def wait_write(t):
            for cb in range(D // 8):
                pltpu.make_async_copy(trv.at[cb],
                                      out_hbm.at[t, cb, pl.ds(sb0, JW)],
                                      ws).wait()

        def transpose(p):
            def jj_body(jj, carry):
                rvecs = [slvecs[b] + jj * 128 for b in range(8)]
                for cb in range(D // 8):
                    for cl in range(8):
                        cvec = lane * 0 + (cb * 8 + cl)
                        vals = [plsc.load_gather(rows[p], [rvecs[b], cvec])
                                for b in range(8)]
                        for b in range(8):
                            trv[cb, jj, cl, pl.ds(b * 16, 16)] = vals[b]
                return carry
            lax.fori_loop(0, JW, jj_body, 0)

        load_idx(0, 0)
        start_gather(0)

        def body(i, carry):
            for p in range(2):
                t = 2 * i + p

                @pl.when(t + 1 < T)
                def _():
                    load_idx(t + 1, 1 - p)
                    start_gather(1 - p)

                wait_gather(p)

                @pl.when(t >= 1)
                def _():
                    wait_write(t - 1)

                transpose(p)
                start_write(t)
            return carry

        lax.fori_loop(0, T // 2, body, 0)
        wait_write(T - 1)

    return k(idx_ts, weight)


def kernel(token_ids, weight):
    S, T = token_ids.shape
    D = weight.shape[1]
    idx_ts = token_ids.T.astype(jnp.int32)
    a = _sc_gather(idx_ts, weight, T)
    return a.transpose(2, 4, 0, 1, 3).reshape(S, T, D)


# speedup vs baseline: 1.1265x; 1.0046x over previous
import functools

import jax
import jax.numpy as jnp
from jax import lax
from jax.experimental import pallas as pl
from jax.experimental.pallas import tpu as pltpu
from jax.experimental.pallas import tpu_sc as plsc

_NUM_CORES = 2
_NUM_SUBCORES = 16
_NUM_WORKERS = _NUM_CORES * _NUM_SUBCORES


@functools.partial(jax.jit, static_argnums=(2,))
def _sc_gather(idx_ts, weight, T):
    S = idx_ts.shape[1]
    D = weight.shape[1]
    CB = D // 8
    SB = S // 128
    JW = SB // _NUM_WORKERS
    R = JW * 128
    mesh = plsc.VectorSubcoreMesh(core_axis_name="c", subcore_axis_name="s")

    @functools.partial(
        pl.kernel,
        mesh=mesh,
        out_type=jax.ShapeDtypeStruct((T, CB, SB * 8, 128), jnp.float32),
        scratch_types=[
            pltpu.VMEM((R,), jnp.int32),
            pltpu.VMEM((R,), jnp.int32),
            pltpu.VMEM((R, D), jnp.float32),
            pltpu.VMEM((R, D), jnp.float32),
            pltpu.VMEM((CB, JW * 8, 128), jnp.float32),
            pltpu.SemaphoreType.DMA,
            pltpu.SemaphoreType.DMA,
            pltpu.SemaphoreType.DMA,
        ],
        compiler_params=pltpu.CompilerParams(
            use_tc_tiling_on_sc=False, needs_layout_passes=False),
    )
    def k(idx_hbm, table_hbm, out_hbm, ib0, ib1, rows0, rows1, trv,
          g0, g1, ws):
        wid = lax.axis_index("s") * _NUM_CORES + lax.axis_index("c")
        sb0 = wid * JW
        ibuf = (ib0, ib1)
        rows = (rows0, rows1)
        gsem = (g0, g1)

        lane = lax.iota(jnp.int32, 16)
        slvecs = [lane + 16 * b for b in range(8)]

        def load_idx(t, p):
            pltpu.sync_copy(idx_hbm.at[t, pl.ds(sb0 * 128, R)], ibuf[p])

        def start_gather(p):
            pltpu.async_copy(table_hbm.at[ibuf[p]], rows[p], gsem[p])

        def wait_gather(p):
            pltpu.make_async_copy(table_hbm.at[ibuf[p]], rows[p],
                                  gsem[p]).wait()

        def start_write(t):
            for cb in range(CB):
                pltpu.async_copy(trv.at[cb],
                                 out_hbm.at[t, cb, pl.ds(sb0 * 8, JW * 8)],
                                 ws)

        def wait_write(t):
            for cb in range(CB):
                pltpu.make_async_copy(trv.at[cb],
                                      out_hbm.at[t, cb,
                                                 pl.ds(sb0 * 8, JW * 8)],
                                      ws).wait()

        def transpose(p):
            def jj_body(jj, carry):
                rvecs = [slvecs[b] + jj * 128 for b in range(8)]
                jj8 = jj * 8
                for c0 in range(0, D, 2):
                    qa = (lane + c0) & (D - 1)
                    qb = (lane + (c0 + 1)) & (D - 1)
                    idxs = []
                    for q in (qa, qb):
                        idxs.append((q >> 3, jj8 + (q & 7)))
                    va = [plsc.load_gather(rows[p], [rvecs[b], qa])
                          for b in range(8)]
                    vb = [plsc.load_gather(rows[p], [rvecs[b], qb])
                          for b in range(8)]
                    for b in range(8):
                        plsc.store_scatter(
                            trv, [idxs[0][0], idxs[0][1], slvecs[b]], va[b])
                        plsc.store_scatter(
                            trv, [idxs[1][0], idxs[1][1], slvecs[b]], vb[b])
                return carry
            lax.fori_loop(0, JW, jj_body, 0)

        load_idx(0, 0)
        start_gather(0)

        def body(i, carry):
            for p in range(2):
                t = 2 * i + p

                @pl.when(t + 1 < T)
                def _():
                    load_idx(t + 1, 1 - p)
                    start_gather(1 - p)

                wait_gather(p)

                @pl.when(t >= 1)
                def _():
                    wait_write(t - 1)

                transpose(p)
                start_write(t)
            return carry

        lax.fori_loop(0, T // 2, body, 0)
        wait_write(T - 1)

    return k(idx_ts, weight)


def kernel(token_ids, weight):
    S, T = token_ids.shape
    D = weight.shape[1]
    idx_ts = token_ids.T.astype(jnp.int32)
    a = _sc_gather(idx_ts, weight, T)
    a = a.reshape(T, D // 8, S // 128, 8, 128)
    return a.transpose(2, 4, 0, 1, 3).reshape(S, T, D)


# speedup vs baseline: 1.4613x; 1.2972x over previous
import functools

import jax
import jax.numpy as jnp
from jax import lax
from jax.experimental import pallas as pl
from jax.experimental.pallas import tpu as pltpu
from jax.experimental.pallas import tpu_sc as plsc

_NUM_CORES = 2
_NUM_SUBCORES = 16
_NUM_WORKERS = _NUM_CORES * _NUM_SUBCORES


@functools.partial(jax.jit, static_argnums=(2, 3, 4))
def _sc_gather(idx_flat, weight, B, D, C):
    b_per_w = B // _NUM_WORKERS
    n_chunks = b_per_w // C
    assert n_chunks * C == b_per_w and n_chunks % 2 == 0
    mesh = plsc.VectorSubcoreMesh(core_axis_name="c", subcore_axis_name="s")

    @functools.partial(
        pl.kernel,
        mesh=mesh,
        out_type=jax.ShapeDtypeStruct((B, D), jnp.float32),
        scratch_types=[
            pltpu.VMEM((b_per_w,), jnp.int32),
            pltpu.VMEM((C, D), jnp.float32),
            pltpu.VMEM((C, D), jnp.float32),
            pltpu.SemaphoreType.DMA,
            pltpu.SemaphoreType.DMA,
            pltpu.SemaphoreType.DMA,
            pltpu.SemaphoreType.DMA,
        ],
        compiler_params=pltpu.CompilerParams(use_tc_tiling_on_sc=False),
    )
    def k(idx_hbm, table_hbm, out_hbm, idx_v, rows0, rows1, g0, g1, s0, s1):
        wid = lax.axis_index("s") * _NUM_CORES + lax.axis_index("c")
        base = wid * b_per_w
        rows = (rows0, rows1)
        gsem = (g0, g1)
        ssem = (s0, s1)

        pltpu.sync_copy(idx_hbm.at[pl.ds(base, b_per_w)], idx_v)

        def start_gather(g, b):
            pltpu.async_copy(
                table_hbm.at[idx_v.at[pl.ds(g * C, C)]], rows[b], gsem[b])

        def wait_gather(g, b):
            pltpu.make_async_copy(
                table_hbm.at[idx_v.at[pl.ds(g * C, C)]], rows[b], gsem[b]
            ).wait()

        def start_store(g, b):
            pltpu.async_copy(rows[b], out_hbm.at[pl.ds(base + g * C, C)],
                             ssem[b])

        def wait_store(g, b):
            pltpu.make_async_copy(
                rows[b], out_hbm.at[pl.ds(base + g * C, C)], ssem[b]
            ).wait()

        start_gather(0, 0)

        def body(i, carry):
            for b in range(2):
                g = 2 * i + b

                @pl.when(g >= 1)
                def _():
                    wait_store(g - 1, 1 - b)

                @pl.when(g + 1 < n_chunks)
                def _():
                    start_gather(g + 1, 1 - b)

                wait_gather(g, b)
                start_store(g, b)
            return carry

        lax.fori_loop(0, n_chunks // 2, body, 0)
        wait_store(n_chunks - 1, (n_chunks - 1) % 2)

    return k(idx_flat, weight)


def kernel(token_ids, weight):
    S, T = token_ids.shape
    D = weight.shape[1]
    B = S * T
    idx_flat = token_ids.reshape(B).astype(jnp.int32)
    out = _sc_gather(idx_flat, weight, B, D, 800)
    return out.reshape(S, T, D)


# speedup vs baseline: 1.9255x; 1.3177x over previous
import functools

import jax
import jax.numpy as jnp
from jax import lax
from jax.experimental import pallas as pl
from jax.experimental.pallas import tpu as pltpu
from jax.experimental.pallas import tpu_sc as plsc

_NUM_CORES = 2
_NUM_SUBCORES = 16
_NUM_WORKERS = _NUM_CORES * _NUM_SUBCORES


@functools.partial(jax.jit, static_argnums=(2,))
def _sc_gather(idx_ts, weight, T):
    S = idx_ts.shape[1]
    D = weight.shape[1]
    TT = (T + 7) // 8
    SC = S // (64 * _NUM_WORKERS)
    NU = SC * TT
    R = 8 * 64
    mesh = plsc.VectorSubcoreMesh(core_axis_name="c", subcore_axis_name="s")

    @functools.partial(
        pl.kernel,
        mesh=mesh,
        out_type=jax.ShapeDtypeStruct((S, TT * 8 * 128), jnp.float32),
        scratch_types=[
            pltpu.VMEM((T, 64 * SC), jnp.int32),
            pltpu.VMEM((NU * R,), jnp.int32),
            pltpu.VMEM((R, D), jnp.float32),
            pltpu.VMEM((R, D), jnp.float32),
            pltpu.SemaphoreType.DMA,
            pltpu.SemaphoreType.DMA,
            pltpu.SemaphoreType.DMA,
        ],
        compiler_params=pltpu.CompilerParams(
            use_tc_tiling_on_sc=False, needs_layout_passes=False),
    )
    def k(idx_hbm, table_hbm, out_hbm, panel, aidx, rows0, rows1,
          g0, g1, ws):
        wid = lax.axis_index("s") * _NUM_CORES + lax.axis_index("c")
        rows = (rows0, rows1)
        gsem = (g0, g1)

        pltpu.sync_copy(idx_hbm.at[:, pl.ds(wid * 64 * SC, 64 * SC)], panel)

        def arrange(u, carry):
            th = u // SC
            sb = u % SC
            for tl in range(8):
                trow = jnp.minimum(8 * th + tl, T - 1)
                for kk in range(4):
                    aidx[pl.ds(u * R + tl * 64 + kk * 16, 16)] = (
                        panel[trow, pl.ds(sb * 64 + kk * 16, 16)])
            return carry
        lax.fori_loop(0, NU, arrange, 0)

        def start_gather(u, p):
            pltpu.async_copy(table_hbm.at[aidx.at[pl.ds(u * R, R)]],
                             rows[p], gsem[p])

        def wait_gather(u, p):
            pltpu.make_async_copy(table_hbm.at[aidx.at[pl.ds(u * R, R)]],
                                  rows[p], gsem[p]).wait()

        def write_copy(u, p, tl):
            th = u // SC
            sb = u % SC
            s0 = (wid * SC + sb) * 64
            return pltpu.make_async_copy(
                rows[p].at[pl.ds(tl * 64, 64)],
                out_hbm.at[pl.ds(s0, 64), pl.ds((th * 8 + tl) * 128, D)],
                ws)

        start_gather(0, 0)

        def body(i, carry):
            for p in range(2):
                u = 2 * i + p

                @pl.when(u + 1 < NU)
                def _():
                    start_gather(u + 1, 1 - p)

                wait_gather(u, p)

                write_copy(u, p, 0).start()
                for tl in range(1, 8):
                    write_copy(u, p, tl).start()
                    write_copy(u, p, tl - 1).wait()
                write_copy(u, p, 7).wait()
            return carry

        lax.fori_loop(0, NU // 2, body, 0)

    return k(idx_ts, weight)


def kernel(token_ids, weight):
    S, T = token_ids.shape
    D = weight.shape[1]
    idx_ts = token_ids.T.astype(jnp.int32)
    a = _sc_gather(idx_ts, weight, T)
    return a.reshape(S, 56, 128)[:, :T, :D]
